# baseline (speedup 1.0000x reference)
import functools

import jax
import jax.numpy as jnp
from jax import lax
from jax.experimental import pallas as pl
from jax.experimental.pallas import tpu as pltpu
from jax.experimental.pallas import tpu_sc as plsc

_Q_BLK = 2048
_E_CHUNK = 1024
_LOG2E = 1.4426950408889634


def _vq_tc_body(x_ref, emb_ref, pw_ref, pb_ref,
                idx_ref, loss_ref, embed_ref,
                et_ref, en_ref, col_ref):
    i = pl.program_id(0)
    n_e = emb_ref.shape[0]

    @pl.when(i == 0)
    def _init():
        e = lax.dot_general(emb_ref[...], pw_ref[...],
                            (((1,), (1,)), ((), ())),
                            preferred_element_type=jnp.float32)
        e = e + pb_ref[...]
        pad = embed_ref.shape[1] - e.shape[1]
        embed_ref[...] = jnp.concatenate(
            [e, jnp.zeros((e.shape[0], pad), jnp.float32)], axis=1)
        et_ref[...] = (-2.0 * e).T
        en_ref[...] = jnp.sum(e * e, axis=1)[None, :]
        col_ref[...] = (lax.broadcasted_iota(jnp.int32, (1, n_e), 1)
                        .astype(jnp.float32))
        loss_ref[...] = jnp.zeros_like(loss_ref)

    q = x_ref[...]
    qn = jnp.sum(q * q, axis=1)[:, None]
    big = jnp.float32(n_e)
    l2e = jnp.float32(_LOG2E)
    m = bi = s = None
    for c in range(n_e // _E_CHUNK):
        lo = c * _E_CHUNK
        qe = lax.dot_general(q, et_ref[:, lo:lo + _E_CHUNK],
                             (((1,), (0,)), ((), ())),
                             preferred_element_type=jnp.float32)
        sq = (qn + en_ref[:, lo:lo + _E_CHUNK]) + qe
        d = jnp.sqrt(jnp.maximum(sq, 0.0))
        mc = jnp.min(d, axis=1, keepdims=True)
        ic = jnp.min(jnp.where(d == mc, col_ref[:, lo:lo + _E_CHUNK], big),
                     axis=1, keepdims=True)
        sc = jnp.sum(jnp.exp2((mc - d) * l2e), axis=1, keepdims=True)
        if c == 0:
            m, bi, s = mc, ic, sc
        else:
            mn = jnp.minimum(m, mc)
            bi = jnp.where(mc < m, ic, bi)
            s = (s * jnp.exp2((mn - m) * l2e)
                 + sc * jnp.exp2((mn - mc) * l2e))
            m = mn
    idx_ref[0, 0, :] = bi[:, 0].astype(jnp.int32)
    loss_ref[...] += jnp.sum(jnp.log(s)).reshape(1, 1)


def _vq_tc(xf, embedding, proj_w, proj_b):
    n_tok, c = xf.shape
    n_e = embedding.shape[0]
    n_blk = n_tok // _Q_BLK
    return pl.pallas_call(
        _vq_tc_body,
        grid=(n_blk,),
        in_specs=[
            pl.BlockSpec((_Q_BLK, c), lambda i: (i, 0)),
            pl.BlockSpec((n_e, c), lambda i: (0, 0)),
            pl.BlockSpec((c, c), lambda i: (0, 0)),
            pl.BlockSpec((1, c), lambda i: (0, 0)),
        ],
        out_specs=(
            pl.BlockSpec((1, 1, _Q_BLK), lambda i: (i, 0, 0)),
            pl.BlockSpec((1, 1), lambda i: (0, 0)),
            pl.BlockSpec((n_e, 128), lambda i: (0, 0)),
        ),
        scratch_shapes=[
            pltpu.VMEM((c, n_e), jnp.float32),
            pltpu.VMEM((1, n_e), jnp.float32),
            pltpu.VMEM((1, n_e), jnp.float32),
        ],
        out_shape=(
            jax.ShapeDtypeStruct((n_blk, 1, _Q_BLK), jnp.int32),
            jax.ShapeDtypeStruct((1, 1), jnp.float32),
            jax.ShapeDtypeStruct((n_e, 128), jnp.float32),
        ),
    )(xf, embedding, proj_w, proj_b.reshape(1, c))


def _sc_gather(table, idx):
    info = plsc.get_sparse_core_info()
    nc, ns = info.num_cores, info.num_subcores
    nw = nc * ns
    b = idx.shape[0]
    d_dim = table.shape[1]
    b_per_w = b // nw
    n_chunk = b_per_w // 128
    idx3 = idx.reshape(nw, n_chunk, 128)
    mesh = plsc.VectorSubcoreMesh(core_axis_name="c", subcore_axis_name="s")

    @functools.partial(
        pl.kernel, mesh=mesh,
        out_type=jax.ShapeDtypeStruct((b, d_dim), jnp.float32),
        scratch_types=[
            pltpu.VMEM((n_chunk, 128), jnp.int32),
            pltpu.VMEM((b_per_w, d_dim), jnp.float32),
            pltpu.SemaphoreType.DMA,
        ],
    )
    def gather_k(table_hbm, idx_hbm, out_hbm, idx_v, rows_v, sem):
        wid = lax.axis_index("s") * nc + lax.axis_index("c")
        base = wid * b_per_w
        pltpu.sync_copy(idx_hbm.at[wid], idx_v)
        descs = [
            pltpu.async_copy(table_hbm.at[idx_v.at[j]],
                             rows_v.at[pl.ds(j * 128, 128)], sem)
            for j in range(n_chunk)
        ]
        for dsc in descs:
            dsc.wait()
        pltpu.sync_copy(rows_v, out_hbm.at[pl.ds(base, b_per_w)])

    return gather_k(table, idx3)


def kernel(x, embedding, proj_w, proj_b):
    bb, tt, cc = x.shape
    xf = x.astype(jnp.float32).reshape(-1, cc)
    idx3, loss11, embed_pad = _vq_tc(xf, embedding, proj_w, proj_b)
    indices = idx3.reshape(-1)
    x_q = _sc_gather(embed_pad, indices)[:, :cc]
    loss = loss11[0, 0] / xf.shape[0]
    x_q_st = xf + lax.stop_gradient(x_q - xf)
    return x_q_st.reshape(bb, tt, cc), loss, indices.reshape(bb, tt, 1)


# speedup vs baseline: 1.2582x; 1.2582x over previous
import functools

import jax
import jax.numpy as jnp
from jax import lax
from jax.experimental import pallas as pl
from jax.experimental.pallas import tpu as pltpu
from jax.experimental.pallas import tpu_sc as plsc

_Q_BLK = 1024
_LOG2E = 1.4426950408889634


def _vq_tc_body(x_ref, emb_ref, pw_ref, pb_ref,
                idx_ref, loss_ref, embed_ref,
                et_ref, en_ref, col_ref):
    i = pl.program_id(0)
    n_e = emb_ref.shape[0]

    @pl.when(i == 0)
    def _init():
        e = lax.dot_general(emb_ref[...], pw_ref[...],
                            (((1,), (1,)), ((), ())),
                            preferred_element_type=jnp.float32)
        e = e + pb_ref[...]
        pad = embed_ref.shape[1] - e.shape[1]
        embed_ref[...] = jnp.concatenate(
            [e, jnp.zeros((e.shape[0], pad), jnp.float32)], axis=1)
        et_ref[...] = (-2.0 * e).T
        en_ref[...] = jnp.sum(e * e, axis=1)[None, :]
        col_ref[...] = (lax.broadcasted_iota(jnp.int32, (1, n_e), 1)
                        .astype(jnp.float32))
        loss_ref[...] = jnp.zeros_like(loss_ref)

    q = x_ref[...]
    qn = jnp.sum(q * q, axis=1)[:, None]
    big = jnp.float32(n_e)
    l2e = jnp.float32(_LOG2E)
    h_w = n_e // 2
    ds = []
    for h in range(2):
        lo = h * h_w
        qe = lax.dot_general(q, et_ref[:, lo:lo + h_w],
                             (((1,), (0,)), ((), ())),
                             preferred_element_type=jnp.float32)
        sq = (qn + en_ref[:, lo:lo + h_w]) + qe
        ds.append(jnp.sqrt(jnp.maximum(sq, 0.0)))
    dmin = jnp.minimum(jnp.min(ds[0], axis=1, keepdims=True),
                       jnp.min(ds[1], axis=1, keepdims=True))
    idxf = big
    s = jnp.zeros((ds[0].shape[0],), jnp.float32)
    for h in range(2):
        lo = h * h_w
        idxf = jnp.minimum(
            idxf, jnp.min(jnp.where(ds[h] == dmin,
                                    col_ref[:, lo:lo + h_w], big), axis=1))
        s = s + jnp.sum(jnp.exp2((dmin - ds[h]) * l2e), axis=1)
    idx_ref[0, 0, :] = idxf.astype(jnp.int32)
    loss_ref[...] += jnp.sum(jnp.log(s)).reshape(1, 1)


def _vq_tc(xf, embedding, proj_w, proj_b):
    n_tok, c = xf.shape
    n_e = embedding.shape[0]
    n_blk = n_tok // _Q_BLK
    return pl.pallas_call(
        _vq_tc_body,
        grid=(n_blk,),
        in_specs=[
            pl.BlockSpec((_Q_BLK, c), lambda i: (i, 0)),
            pl.BlockSpec((n_e, c), lambda i: (0, 0)),
            pl.BlockSpec((c, c), lambda i: (0, 0)),
            pl.BlockSpec((1, c), lambda i: (0, 0)),
        ],
        out_specs=(
            pl.BlockSpec((1, 1, _Q_BLK), lambda i: (i, 0, 0)),
            pl.BlockSpec((1, 1), lambda i: (0, 0)),
            pl.BlockSpec((n_e, 128), lambda i: (0, 0)),
        ),
        scratch_shapes=[
            pltpu.VMEM((c, n_e), jnp.float32),
            pltpu.VMEM((1, n_e), jnp.float32),
            pltpu.VMEM((1, n_e), jnp.float32),
        ],
        out_shape=(
            jax.ShapeDtypeStruct((n_blk, 1, _Q_BLK), jnp.int32),
            jax.ShapeDtypeStruct((1, 1), jnp.float32),
            jax.ShapeDtypeStruct((n_e, 128), jnp.float32),
        ),
    )(xf, embedding, proj_w, proj_b.reshape(1, c))


def _sc_gather(table, idx):
    info = plsc.get_sparse_core_info()
    nc, ns = info.num_cores, info.num_subcores
    nw = nc * ns
    b = idx.shape[0]
    d_dim = table.shape[1]
    b_per_w = b // nw
    n_chunk = b_per_w // 128
    idx3 = idx.reshape(nw, n_chunk, 128)
    mesh = plsc.VectorSubcoreMesh(core_axis_name="c", subcore_axis_name="s")

    @functools.partial(
        pl.kernel, mesh=mesh,
        out_type=jax.ShapeDtypeStruct((b, d_dim), jnp.float32),
        scratch_types=[
            pltpu.VMEM((n_chunk, 128), jnp.int32),
            pltpu.VMEM((b_per_w, d_dim), jnp.float32),
            pltpu.SemaphoreType.DMA,
        ],
    )
    def gather_k(table_hbm, idx_hbm, out_hbm, idx_v, rows_v, sem):
        wid = lax.axis_index("s") * nc + lax.axis_index("c")
        base = wid * b_per_w
        pltpu.sync_copy(idx_hbm.at[wid], idx_v)
        descs = [
            pltpu.async_copy(table_hbm.at[idx_v.at[j]],
                             rows_v.at[pl.ds(j * 128, 128)], sem)
            for j in range(n_chunk)
        ]
        for dsc in descs:
            dsc.wait()
        pltpu.sync_copy(rows_v, out_hbm.at[pl.ds(base, b_per_w)])

    return gather_k(table, idx3)


def kernel(x, embedding, proj_w, proj_b):
    bb, tt, cc = x.shape
    xf = x.astype(jnp.float32).reshape(-1, cc)
    idx3, loss11, embed_pad = _vq_tc(xf, embedding, proj_w, proj_b)
    indices = idx3.reshape(-1)
    x_q = _sc_gather(embed_pad, indices)[:, :cc]
    loss = loss11[0, 0] / xf.shape[0]
    x_q_st = xf + lax.stop_gradient(x_q - xf)
    return x_q_st.reshape(bb, tt, cc), loss, indices.reshape(bb, tt, 1)
